# baseline (speedup 1.0000x reference)
import functools

import jax
import jax.numpy as jnp
from jax.experimental import pallas as pl

N = 10000
E = 160000
D = 64
K = 16
RL = 128
NL = 4
TPOS = RL - K + 1

N_PAD = 10240
NBLK = 128


def _embed_body(reads_ref, wf_ref, b_ref, h_ref):
    r = reads_ref[...]
    wf = wf_ref[...]
    parts = []
    for c in range(4):
        rc = r[:, c, :]
        for k in range(K):
            if k == 0:
                parts.append(rc)
            else:
                parts.append(jnp.concatenate([rc[:, k:], rc[:, :k]], axis=1))
    p2 = jnp.stack(parts, axis=0)
    p2f = p2.reshape(64, NBLK * RL)
    z = jnp.dot(wf, p2f, preferred_element_type=jnp.float32)
    z = z + b_ref[...].reshape(D, 1)
    z = jnp.maximum(z, 0.0)
    t = jax.lax.broadcasted_iota(jnp.int32, (NBLK, RL), 1)
    mask = (t < TPOS).astype(jnp.float32).reshape(1, NBLK * RL)
    z = z * mask
    z = z.reshape(D, NBLK, RL).sum(axis=2) * (1.0 / TPOS)
    h_ref[...] = z.T


def _embed(reads_pad, W_seq, b_seq):
    wf = W_seq.reshape(D, 64)
    grid = N_PAD // NBLK
    return pl.pallas_call(
        _embed_body,
        grid=(grid,),
        in_specs=[
            pl.BlockSpec((NBLK, 4, RL), lambda i: (i, 0, 0)),
            pl.BlockSpec((D, 64), lambda i: (0, 0)),
            pl.BlockSpec((D,), lambda i: (0,)),
        ],
        out_specs=pl.BlockSpec((NBLK, D), lambda i: (i, 0)),
        out_shape=jax.ShapeDtypeStruct((N_PAD, D), jnp.float32),
    )(reads_pad, wf, b_seq)


def kernel(reads, edge_index, overlap_similarity, overlap_length,
           W_seq, b_seq, W_edge, b_edge,
           A, bA, B, bB, C, bC, U, bU, V, bV,
           Wd1, bd1, Wd2, bd2):
    src = edge_index[0]
    dst = edge_index[1]
    reads_pad = jnp.pad(reads, ((0, N_PAD - N), (0, 0), (0, 0)))
    h = _embed(reads_pad, W_seq, b_seq)[:N]
    e = overlap_similarity[:, None] @ W_edge + b_edge
    for i in range(NL):
        Ah = h @ A[i] + bA[i]
        Bh = h @ B[i] + bB[i]
        Vh = h @ V[i] + bV[i]
        e_hat = Ah[src] + Bh[dst] + e @ C[i] + bC[i]
        sigma = jax.nn.sigmoid(e_hat)
        num = jax.ops.segment_sum(sigma * Vh[src], dst, num_segments=N)
        den = jax.ops.segment_sum(sigma, dst, num_segments=N)
        h = h + jax.nn.relu(h @ U[i] + bU[i] + num / (den + 1e-6))
        e = e + jax.nn.relu(e_hat)
    x = jnp.concatenate([h[src], h[dst], e], axis=1)
    p = jax.nn.relu(x @ Wd1 + bd1) @ Wd2 + bd2
    return p
